# baseline (speedup 1.0000x reference)
import functools

import jax
import jax.numpy as jnp
from jax import lax
from jax.experimental import pallas as pl
from jax.experimental.pallas import tpu as pltpu
from jax.experimental.pallas import tpu_sc as plsc

_BLOCK_C = 2048
_LANES = 128

_SC_CHUNK = 2048
_SC_NCHUNKS = 24
_SC_QUARTER = _SC_CHUNK * _SC_NCHUNKS
_SC_N = _SC_QUARTER * 4

_R0 = (13, 15, 26, 6)
_R1 = (17, 29, 16, 24)
_KS0 = 0
_KS1 = 42
_KS2 = _KS0 ^ _KS1 ^ 0x1BD11BDA

_TINY = 1.1754943508222875e-38
_LN2 = 0.6931471805599453
_SQRT2 = 1.4142135623730951
_LOG2_C = tuple((2.0 / _LN2) / (2 * k + 1) for k in range(6))


def _rotl(x, r):
    return (x << jnp.uint32(r)) | (x >> jnp.uint32(32 - r))


def _threefry_bits(i):
    ks0 = jnp.uint32(_KS0)
    ks1 = jnp.uint32(_KS1)
    ks2 = jnp.uint32(_KS2)
    x1 = i + ks1
    x0 = x1
    x1 = _rotl(x1, _R0[0]) ^ x0
    for r in _R0[1:]:
        x0 = x0 + x1
        x1 = _rotl(x1, r) ^ x0
    x0 = x0 + ks1
    x1 = x1 + jnp.uint32(_KS2 + 1)
    for r in _R1:
        x0 = x0 + x1
        x1 = _rotl(x1, r) ^ x0
    x0 = x0 + ks2
    x1 = x1 + jnp.uint32(_KS0 + 2)
    for r in _R0:
        x0 = x0 + x1
        x1 = _rotl(x1, r) ^ x0
    x0 = x0 + ks0
    x1 = x1 + jnp.uint32(_KS1 + 3)
    for r in _R1:
        x0 = x0 + x1
        x1 = _rotl(x1, r) ^ x0
    x0 = x0 + ks1
    x1 = x1 + jnp.uint32(_KS2 + 4)
    for r in _R0:
        x0 = x0 + x1
        x1 = _rotl(x1, r) ^ x0
    x0 = x0 + ks2
    x1 = x1 + jnp.uint32(_KS0 + 5)
    return x0 ^ x1


def _uniform_from_bits(bits):
    fb = (bits >> jnp.uint32(9)) | jnp.uint32(0x3F800000)
    f = lax.bitcast_convert_type(fb, jnp.float32) - jnp.float32(1.0)
    return jnp.maximum(f, jnp.float32(_TINY))


def _tc_kernel(logits_ref, val_ref, idx_ref, i_ref, max_ref, lidx_ref, *,
               stride, c0, block_c, nblocks):
    j = pl.program_id(0)
    nrows = logits_ref.shape[0]

    @pl.when(j == 0)
    def _init():
        row = lax.broadcasted_iota(jnp.uint32, (nrows, block_c), 0)
        colv = lax.broadcasted_iota(jnp.uint32, (nrows, block_c), 1)
        i_ref[...] = row * jnp.uint32(stride) + colv + jnp.uint32(c0)
        max_ref[...] = jnp.full_like(max_ref, -jnp.inf)
        lidx_ref[...] = jnp.zeros_like(lidx_ref)

    i = i_ref[...]
    u = _uniform_from_bits(_threefry_bits(i))
    phi = logits_ref[...] - jnp.log(-jnp.log(u))
    ilin = i.astype(jnp.int32)

    run_max = max_ref[...]
    run_idx = lidx_ref[...]
    for k in range(block_c // _LANES):
        sl = slice(k * _LANES, (k + 1) * _LANES)
        chunk = phi[:, sl]
        better = chunk > run_max
        run_max = jnp.where(better, chunk, run_max)
        run_idx = jnp.where(better, ilin[:, sl], run_idx)
    max_ref[...] = run_max
    lidx_ref[...] = run_idx
    i_ref[...] = i + jnp.uint32(block_c)

    @pl.when(j == nblocks - 1)
    def _finish():
        val_ref[...] = max_ref[...]
        idx_ref[...] = lidx_ref[...]


def _tc_partial(logits, c0, c1):
    nrows, ncols = logits.shape
    block_c = _BLOCK_C
    assert c0 % block_c == 0 and (c1 - c0) % block_c == 0
    j0 = c0 // block_c
    nblocks = (c1 - c0) // block_c
    return pl.pallas_call(
        functools.partial(_tc_kernel, stride=ncols, c0=c0,
                          block_c=block_c, nblocks=nblocks),
        grid=(nblocks,),
        in_specs=[pl.BlockSpec((nrows, block_c), lambda j: (0, j + j0))],
        out_specs=[pl.BlockSpec((nrows, _LANES), lambda j: (0, 0)),
                   pl.BlockSpec((nrows, _LANES), lambda j: (0, 0))],
        out_shape=[jax.ShapeDtypeStruct((nrows, _LANES), jnp.float32),
                   jax.ShapeDtypeStruct((nrows, _LANES), jnp.int32)],
        scratch_shapes=[
            pltpu.VMEM((nrows, block_c), jnp.uint32),
            pltpu.VMEM((nrows, _LANES), jnp.float32),
            pltpu.VMEM((nrows, _LANES), jnp.int32),
        ],
    )(logits)


def _sc_log2(x):
    b = lax.bitcast_convert_type(x, jnp.int32)
    e = (b >> jnp.int32(23)) - jnp.int32(127)
    m = lax.bitcast_convert_type(
        (b & jnp.int32(0x7FFFFF)) | jnp.int32(0x3F800000), jnp.float32)
    big = m >= jnp.float32(_SQRT2)
    m = jnp.where(big, m * jnp.float32(0.5), m)
    e = jnp.where(big, e + jnp.int32(1), e)
    z = (m - jnp.float32(1.0)) / (m + jnp.float32(1.0))
    z2 = z * z
    p = jnp.float32(_LOG2_C[5])
    for c in _LOG2_C[4::-1]:
        p = p * z2 + jnp.float32(c)
    return e.astype(jnp.float32) + z * p


def _sc_kernel(logits_hbm, val_hbm, idx_hbm, buf, vval, vidx, sem0, sem1,
               *, stride):
    c = lax.axis_index("c")
    s = lax.axis_index("s")
    wid = s * 2 + c
    band = wid % 8
    quarter = wid // 8
    qbase = quarter * _SC_QUARTER
    row0 = pl.multiple_of(band * 8, 8)
    lane = lax.iota(jnp.int32, 16).astype(jnp.uint32)

    def src(k):
        colstart = pl.multiple_of(qbase + k * _SC_CHUNK, _SC_CHUNK)
        return logits_hbm.at[pl.ds(row0, 8), pl.ds(colstart, _SC_CHUNK)]

    pltpu.async_copy(src(0), buf.at[0], sem0)

    def chunk_body(k, carry):
        p = k & 1
        colstart = pl.multiple_of(qbase + k * _SC_CHUNK, _SC_CHUNK)

        @pl.when((k + 1 < _SC_NCHUNKS) & (p == 0))
        def _start_odd():
            pltpu.async_copy(src(k + 1), buf.at[1], sem1)

        @pl.when((k + 1 < _SC_NCHUNKS) & (p == 1))
        def _start_even():
            pltpu.async_copy(src(k + 1), buf.at[0], sem0)

        @pl.when(p == 0)
        def _wait_even():
            pltpu.make_async_copy(src(k), buf.at[0], sem0).wait()

        @pl.when(p == 1)
        def _wait_odd():
            pltpu.make_async_copy(src(k), buf.at[1], sem1).wait()

        out = []
        for rr in range(8):
            run_max, run_idx = carry[2 * rr], carry[2 * rr + 1]
            ibase = ((row0 + rr) * stride + colstart).astype(jnp.uint32)

            def vec_body(v, c2, rr=rr, ibase=ibase):
                rm, ri = c2
                lvec = buf[p, rr, pl.ds(v * 16, 16)]
                i = ibase + (v * 16).astype(jnp.uint32) + lane
                u = _uniform_from_bits(_threefry_bits(i))
                inner = jnp.float32(-_LN2) * _sc_log2(u)
                phi = lvec - jnp.float32(_LN2) * _sc_log2(inner)
                better = phi > rm
                rm = jnp.where(better, phi, rm)
                ri = jnp.where(better, lax.bitcast_convert_type(i, jnp.int32), ri)
                return rm, ri

            run_max, run_idx = lax.fori_loop(
                0, _SC_CHUNK // 16, vec_body, (run_max, run_idx), unroll=4)
            out.extend([run_max, run_idx])
        return tuple(out)

    init = []
    for _ in range(8):
        init.extend([jnp.full((16,), -jnp.inf, jnp.float32),
                     jnp.zeros((16,), jnp.int32)])
    final = lax.fori_loop(0, _SC_NCHUNKS, chunk_body, tuple(init))
    for rr in range(8):
        vval[rr, :] = final[2 * rr]
        vidx[rr, :] = final[2 * rr + 1]
    pltpu.sync_copy(vval, val_hbm.at[wid])
    pltpu.sync_copy(vidx, idx_hbm.at[wid])


def _sc_partial(logits):
    nrows, ncols = logits.shape
    mesh = plsc.VectorSubcoreMesh(core_axis_name="c", subcore_axis_name="s",
                                  num_cores=2, num_subcores=16)
    return pl.kernel(
        functools.partial(_sc_kernel, stride=ncols),
        out_type=[jax.ShapeDtypeStruct((32, 8, 16), jnp.float32),
                  jax.ShapeDtypeStruct((32, 8, 16), jnp.int32)],
        mesh=mesh,
        scratch_types=[
            pltpu.VMEM((2, 8, _SC_CHUNK), jnp.float32),
            pltpu.VMEM((8, 16), jnp.float32),
            pltpu.VMEM((8, 16), jnp.int32),
            pltpu.SemaphoreType.DMA,
            pltpu.SemaphoreType.DMA,
        ],
    )(logits)


def _merge_kernel(tcv_ref, tci_ref, scv_ref, sci_ref, tail_ref, out_ref, *,
                  stride, tail0):
    nrows = tcv_ref.shape[0]
    ntail = tail_ref.shape[1]
    row = lax.broadcasted_iota(jnp.uint32, (nrows, ntail), 0)
    colv = lax.broadcasted_iota(jnp.uint32, (nrows, ntail), 1)
    ti = row * jnp.uint32(stride) + colv + jnp.uint32(tail0)
    u = _uniform_from_bits(_threefry_bits(ti))
    tphi = tail_ref[...] - jnp.log(-jnp.log(u))
    tlin = ti.astype(jnp.int32)
    tv = jnp.full((nrows, _LANES), -jnp.inf, jnp.float32)
    tix = jnp.zeros((nrows, _LANES), jnp.int32)
    for k in range(ntail // _LANES):
        sl = slice(k * _LANES, (k + 1) * _LANES)
        chunk = tphi[:, sl]
        better = chunk > tv
        tv = jnp.where(better, chunk, tv)
        tix = jnp.where(better, tlin[:, sl], tix)

    tcv = tcv_ref[...]
    tci = tci_ref[...]
    scv = scv_ref[...]
    sci = sci_ref[...]
    big = jnp.int32(0x7FFFFFFF)
    row_max = jnp.maximum(jnp.max(tcv, axis=1, keepdims=True),
                          jnp.max(scv, axis=1, keepdims=True))
    row_max = jnp.maximum(row_max, jnp.max(tv, axis=1, keepdims=True))
    c1 = jnp.min(jnp.where(tcv == row_max, tci, big), axis=1, keepdims=True)
    c2 = jnp.min(jnp.where(scv == row_max, sci, big), axis=1, keepdims=True)
    c3 = jnp.min(jnp.where(tv == row_max, tix, big), axis=1, keepdims=True)
    best_lin = jnp.minimum(jnp.minimum(c1, c2), c3)
    row1 = lax.broadcasted_iota(jnp.int32, (nrows, 1), 0)
    out_ref[...] = jnp.broadcast_to(best_lin - row1 * jnp.int32(stride),
                                    out_ref.shape)


def _merge(tcv, tci, scv, sci, tail, stride, tail0):
    nrows = tcv.shape[0]
    return pl.pallas_call(
        functools.partial(_merge_kernel, stride=stride, tail0=tail0),
        out_shape=jax.ShapeDtypeStruct((nrows, _LANES), jnp.int32),
    )(tcv, tci, scv, sci, tail)


def _sc_to_rows(x, fill):
    x = x.reshape(4, 8, 8, 16).transpose(1, 2, 0, 3).reshape(64, 64)
    pad = jnp.full((64, 64), fill, x.dtype)
    return jnp.concatenate([x, pad], axis=1)


@jax.jit
def kernel(logits):
    nrows, ncols = logits.shape
    c1 = (ncols // _BLOCK_C) * _BLOCK_C
    ntail_pad = ((ncols - c1 + _LANES - 1) // _LANES) * _LANES
    tail = jnp.pad(logits[:, c1:], ((0, 0), (0, ntail_pad - (ncols - c1))),
                   constant_values=-jnp.inf)
    tcv, tci = _tc_partial(logits, _SC_N, c1)
    scv, sci = _sc_partial(logits)
    scv_p = _sc_to_rows(scv, -jnp.inf)
    sci_p = _sc_to_rows(sci, 0x7FFFFFFF)
    out = _merge(tcv, tci, scv_p, sci_p, tail, ncols, c1)
    return out[:, :1]


# speedup vs baseline: 1.0076x; 1.0076x over previous
import functools

import jax
import jax.numpy as jnp
from jax import lax
from jax.experimental import pallas as pl
from jax.experimental.pallas import tpu as pltpu
from jax.experimental.pallas import tpu_sc as plsc

_BLOCK_C = 2048
_LANES = 128

_SC_CHUNK = 2048
_SC_NCHUNKS = 25
_SC_QUARTER = _SC_CHUNK * _SC_NCHUNKS
_SC_N = _SC_QUARTER * 4

_R0 = (13, 15, 26, 6)
_R1 = (17, 29, 16, 24)
_KS0 = 0
_KS1 = 42
_KS2 = _KS0 ^ _KS1 ^ 0x1BD11BDA

_TINY = 1.1754943508222875e-38
_LN2 = 0.6931471805599453
_SQRT2 = 1.4142135623730951
_LOG2_C = tuple((2.0 / _LN2) / (2 * k + 1) for k in range(6))


def _rotl(x, r):
    return (x << jnp.uint32(r)) | (x >> jnp.uint32(32 - r))


def _threefry_bits(i):
    ks0 = jnp.uint32(_KS0)
    ks1 = jnp.uint32(_KS1)
    ks2 = jnp.uint32(_KS2)
    x1 = i + ks1
    x0 = x1
    x1 = _rotl(x1, _R0[0]) ^ x0
    for r in _R0[1:]:
        x0 = x0 + x1
        x1 = _rotl(x1, r) ^ x0
    x0 = x0 + ks1
    x1 = x1 + jnp.uint32(_KS2 + 1)
    for r in _R1:
        x0 = x0 + x1
        x1 = _rotl(x1, r) ^ x0
    x0 = x0 + ks2
    x1 = x1 + jnp.uint32(_KS0 + 2)
    for r in _R0:
        x0 = x0 + x1
        x1 = _rotl(x1, r) ^ x0
    x0 = x0 + ks0
    x1 = x1 + jnp.uint32(_KS1 + 3)
    for r in _R1:
        x0 = x0 + x1
        x1 = _rotl(x1, r) ^ x0
    x0 = x0 + ks1
    x1 = x1 + jnp.uint32(_KS2 + 4)
    for r in _R0:
        x0 = x0 + x1
        x1 = _rotl(x1, r) ^ x0
    x0 = x0 + ks2
    x1 = x1 + jnp.uint32(_KS0 + 5)
    return x0 ^ x1


def _uniform_from_bits(bits):
    fb = (bits >> jnp.uint32(9)) | jnp.uint32(0x3F800000)
    f = lax.bitcast_convert_type(fb, jnp.float32) - jnp.float32(1.0)
    return jnp.maximum(f, jnp.float32(_TINY))


def _tc_kernel(logits_ref, val_ref, idx_ref, i_ref, max_ref, lidx_ref, *,
               stride, c0, block_c, nblocks):
    j = pl.program_id(0)
    nrows = logits_ref.shape[0]

    @pl.when(j == 0)
    def _init():
        row = lax.broadcasted_iota(jnp.uint32, (nrows, block_c), 0)
        colv = lax.broadcasted_iota(jnp.uint32, (nrows, block_c), 1)
        i_ref[...] = row * jnp.uint32(stride) + colv + jnp.uint32(c0)
        max_ref[...] = jnp.full_like(max_ref, -jnp.inf)
        lidx_ref[...] = jnp.zeros_like(lidx_ref)

    i = i_ref[...]
    u = _uniform_from_bits(_threefry_bits(i))
    phi = logits_ref[...] - jnp.log(-jnp.log(u))
    ilin = i.astype(jnp.int32)

    run_max = max_ref[...]
    run_idx = lidx_ref[...]
    for k in range(block_c // _LANES):
        sl = slice(k * _LANES, (k + 1) * _LANES)
        chunk = phi[:, sl]
        better = chunk > run_max
        run_max = jnp.where(better, chunk, run_max)
        run_idx = jnp.where(better, ilin[:, sl], run_idx)
    max_ref[...] = run_max
    lidx_ref[...] = run_idx
    i_ref[...] = i + jnp.uint32(block_c)

    @pl.when(j == nblocks - 1)
    def _finish():
        val_ref[...] = max_ref[...]
        idx_ref[...] = lidx_ref[...]


def _tc_partial(logits, c0, c1):
    nrows, ncols = logits.shape
    block_c = _BLOCK_C
    assert c0 % block_c == 0 and (c1 - c0) % block_c == 0
    j0 = c0 // block_c
    nblocks = (c1 - c0) // block_c
    return pl.pallas_call(
        functools.partial(_tc_kernel, stride=ncols, c0=c0,
                          block_c=block_c, nblocks=nblocks),
        grid=(nblocks,),
        in_specs=[pl.BlockSpec((nrows, block_c), lambda j: (0, j + j0))],
        out_specs=[pl.BlockSpec((nrows, _LANES), lambda j: (0, 0)),
                   pl.BlockSpec((nrows, _LANES), lambda j: (0, 0))],
        out_shape=[jax.ShapeDtypeStruct((nrows, _LANES), jnp.float32),
                   jax.ShapeDtypeStruct((nrows, _LANES), jnp.int32)],
        scratch_shapes=[
            pltpu.VMEM((nrows, block_c), jnp.uint32),
            pltpu.VMEM((nrows, _LANES), jnp.float32),
            pltpu.VMEM((nrows, _LANES), jnp.int32),
        ],
    )(logits)


def _sc_log2(x):
    b = lax.bitcast_convert_type(x, jnp.int32)
    e = (b >> jnp.int32(23)) - jnp.int32(127)
    m = lax.bitcast_convert_type(
        (b & jnp.int32(0x7FFFFF)) | jnp.int32(0x3F800000), jnp.float32)
    big = m >= jnp.float32(_SQRT2)
    m = jnp.where(big, m * jnp.float32(0.5), m)
    e = jnp.where(big, e + jnp.int32(1), e)
    z = (m - jnp.float32(1.0)) / (m + jnp.float32(1.0))
    z2 = z * z
    p = jnp.float32(_LOG2_C[5])
    for c in _LOG2_C[4::-1]:
        p = p * z2 + jnp.float32(c)
    return e.astype(jnp.float32) + z * p


def _sc_kernel(logits_hbm, val_hbm, idx_hbm, buf, vval, vidx, sem0, sem1,
               *, stride):
    c = lax.axis_index("c")
    s = lax.axis_index("s")
    wid = s * 2 + c
    band = wid % 8
    quarter = wid // 8
    qbase = quarter * _SC_QUARTER
    row0 = pl.multiple_of(band * 8, 8)
    lane = lax.iota(jnp.int32, 16).astype(jnp.uint32)

    def src(k):
        colstart = pl.multiple_of(qbase + k * _SC_CHUNK, _SC_CHUNK)
        return logits_hbm.at[pl.ds(row0, 8), pl.ds(colstart, _SC_CHUNK)]

    pltpu.async_copy(src(0), buf.at[0], sem0)

    def chunk_body(k, carry):
        p = k & 1
        colstart = pl.multiple_of(qbase + k * _SC_CHUNK, _SC_CHUNK)

        @pl.when((k + 1 < _SC_NCHUNKS) & (p == 0))
        def _start_odd():
            pltpu.async_copy(src(k + 1), buf.at[1], sem1)

        @pl.when((k + 1 < _SC_NCHUNKS) & (p == 1))
        def _start_even():
            pltpu.async_copy(src(k + 1), buf.at[0], sem0)

        @pl.when(p == 0)
        def _wait_even():
            pltpu.make_async_copy(src(k), buf.at[0], sem0).wait()

        @pl.when(p == 1)
        def _wait_odd():
            pltpu.make_async_copy(src(k), buf.at[1], sem1).wait()

        out = []
        for rr in range(8):
            run_max, run_idx = carry[2 * rr], carry[2 * rr + 1]
            ibase = ((row0 + rr) * stride + colstart).astype(jnp.uint32)

            def vec_body(v, c2, rr=rr, ibase=ibase):
                rm, ri = c2
                lvec = buf[p, rr, pl.ds(v * 16, 16)]
                i = ibase + (v * 16).astype(jnp.uint32) + lane
                u = _uniform_from_bits(_threefry_bits(i))
                inner = jnp.float32(-_LN2) * _sc_log2(u)
                phi = lvec - jnp.float32(_LN2) * _sc_log2(inner)
                better = phi > rm
                rm = jnp.where(better, phi, rm)
                ri = jnp.where(better, lax.bitcast_convert_type(i, jnp.int32), ri)
                return rm, ri

            run_max, run_idx = lax.fori_loop(
                0, _SC_CHUNK // 16, vec_body, (run_max, run_idx), unroll=4)
            out.extend([run_max, run_idx])
        return tuple(out)

    init = []
    for _ in range(8):
        init.extend([jnp.full((16,), -jnp.inf, jnp.float32),
                     jnp.zeros((16,), jnp.int32)])
    final = lax.fori_loop(0, _SC_NCHUNKS, chunk_body, tuple(init))
    for rr in range(8):
        vval[rr, :] = final[2 * rr]
        vidx[rr, :] = final[2 * rr + 1]
    pltpu.sync_copy(vval, val_hbm.at[wid])
    pltpu.sync_copy(vidx, idx_hbm.at[wid])


def _sc_partial(logits):
    nrows, ncols = logits.shape
    mesh = plsc.VectorSubcoreMesh(core_axis_name="c", subcore_axis_name="s",
                                  num_cores=2, num_subcores=16)
    return pl.kernel(
        functools.partial(_sc_kernel, stride=ncols),
        out_type=[jax.ShapeDtypeStruct((32, 8, 16), jnp.float32),
                  jax.ShapeDtypeStruct((32, 8, 16), jnp.int32)],
        mesh=mesh,
        scratch_types=[
            pltpu.VMEM((2, 8, _SC_CHUNK), jnp.float32),
            pltpu.VMEM((8, 16), jnp.float32),
            pltpu.VMEM((8, 16), jnp.int32),
            pltpu.SemaphoreType.DMA,
            pltpu.SemaphoreType.DMA,
        ],
    )(logits)


def _merge_kernel(tcv_ref, tci_ref, scv_ref, sci_ref, tail_ref, out_ref, *,
                  stride, tail0):
    nrows = tcv_ref.shape[0]
    ntail = tail_ref.shape[1]
    row = lax.broadcasted_iota(jnp.uint32, (nrows, ntail), 0)
    colv = lax.broadcasted_iota(jnp.uint32, (nrows, ntail), 1)
    ti = row * jnp.uint32(stride) + colv + jnp.uint32(tail0)
    u = _uniform_from_bits(_threefry_bits(ti))
    tphi = tail_ref[...] - jnp.log(-jnp.log(u))
    tlin = ti.astype(jnp.int32)
    tv = jnp.full((nrows, _LANES), -jnp.inf, jnp.float32)
    tix = jnp.zeros((nrows, _LANES), jnp.int32)
    for k in range(ntail // _LANES):
        sl = slice(k * _LANES, (k + 1) * _LANES)
        chunk = tphi[:, sl]
        better = chunk > tv
        tv = jnp.where(better, chunk, tv)
        tix = jnp.where(better, tlin[:, sl], tix)

    tcv = tcv_ref[...]
    tci = tci_ref[...]
    scv = scv_ref[...]
    sci = sci_ref[...]
    big = jnp.int32(0x7FFFFFFF)
    row_max = jnp.maximum(jnp.max(tcv, axis=1, keepdims=True),
                          jnp.max(scv, axis=1, keepdims=True))
    row_max = jnp.maximum(row_max, jnp.max(tv, axis=1, keepdims=True))
    c1 = jnp.min(jnp.where(tcv == row_max, tci, big), axis=1, keepdims=True)
    c2 = jnp.min(jnp.where(scv == row_max, sci, big), axis=1, keepdims=True)
    c3 = jnp.min(jnp.where(tv == row_max, tix, big), axis=1, keepdims=True)
    best_lin = jnp.minimum(jnp.minimum(c1, c2), c3)
    row1 = lax.broadcasted_iota(jnp.int32, (nrows, 1), 0)
    out_ref[...] = jnp.broadcast_to(best_lin - row1 * jnp.int32(stride),
                                    out_ref.shape)


def _merge(tcv, tci, scv, sci, tail, stride, tail0):
    nrows = tcv.shape[0]
    return pl.pallas_call(
        functools.partial(_merge_kernel, stride=stride, tail0=tail0),
        out_shape=jax.ShapeDtypeStruct((nrows, _LANES), jnp.int32),
    )(tcv, tci, scv, sci, tail)


def _sc_to_rows(x, fill):
    x = x.reshape(4, 8, 8, 16).transpose(1, 2, 0, 3).reshape(64, 64)
    pad = jnp.full((64, 64), fill, x.dtype)
    return jnp.concatenate([x, pad], axis=1)


@jax.jit
def kernel(logits):
    nrows, ncols = logits.shape
    c1 = (ncols // _BLOCK_C) * _BLOCK_C
    ntail_pad = ((ncols - c1 + _LANES - 1) // _LANES) * _LANES
    tail = jnp.pad(logits[:, c1:], ((0, 0), (0, ntail_pad - (ncols - c1))),
                   constant_values=-jnp.inf)
    tcv, tci = _tc_partial(logits, _SC_N, c1)
    scv, sci = _sc_partial(logits)
    scv_p = _sc_to_rows(scv, -jnp.inf)
    sci_p = _sc_to_rows(sci, 0x7FFFFFFF)
    out = _merge(tcv, tci, scv_p, sci_p, tail, ncols, c1)
    return out[:, :1]


# speedup vs baseline: 1.0323x; 1.0245x over previous
import functools

import jax
import jax.numpy as jnp
from jax import lax
from jax.experimental import pallas as pl
from jax.experimental.pallas import tpu as pltpu
from jax.experimental.pallas import tpu_sc as plsc

_BLOCK_C = 3072
_LANES = 128

_SC_CHUNK = 2048
_SC_NCHUNKS = 24
_SC_QUARTER = _SC_CHUNK * _SC_NCHUNKS
_SC_N = _SC_QUARTER * 4

_R0 = (13, 15, 26, 6)
_R1 = (17, 29, 16, 24)
_KS0 = 0
_KS1 = 42
_KS2 = _KS0 ^ _KS1 ^ 0x1BD11BDA

_TINY = 1.1754943508222875e-38
_LN2 = 0.6931471805599453
_SQRT2 = 1.4142135623730951
_LOG2_C = tuple((2.0 / _LN2) / (2 * k + 1) for k in range(6))


def _rotl(x, r):
    return (x << jnp.uint32(r)) | (x >> jnp.uint32(32 - r))


def _threefry_bits(i):
    ks0 = jnp.uint32(_KS0)
    ks1 = jnp.uint32(_KS1)
    ks2 = jnp.uint32(_KS2)
    x1 = i + ks1
    x0 = x1
    x1 = _rotl(x1, _R0[0]) ^ x0
    for r in _R0[1:]:
        x0 = x0 + x1
        x1 = _rotl(x1, r) ^ x0
    x0 = x0 + ks1
    x1 = x1 + jnp.uint32(_KS2 + 1)
    for r in _R1:
        x0 = x0 + x1
        x1 = _rotl(x1, r) ^ x0
    x0 = x0 + ks2
    x1 = x1 + jnp.uint32(_KS0 + 2)
    for r in _R0:
        x0 = x0 + x1
        x1 = _rotl(x1, r) ^ x0
    x0 = x0 + ks0
    x1 = x1 + jnp.uint32(_KS1 + 3)
    for r in _R1:
        x0 = x0 + x1
        x1 = _rotl(x1, r) ^ x0
    x0 = x0 + ks1
    x1 = x1 + jnp.uint32(_KS2 + 4)
    for r in _R0:
        x0 = x0 + x1
        x1 = _rotl(x1, r) ^ x0
    x0 = x0 + ks2
    x1 = x1 + jnp.uint32(_KS0 + 5)
    return x0 ^ x1


def _uniform_from_bits(bits):
    fb = (bits >> jnp.uint32(9)) | jnp.uint32(0x3F800000)
    f = lax.bitcast_convert_type(fb, jnp.float32) - jnp.float32(1.0)
    return jnp.maximum(f, jnp.float32(_TINY))


def _tc_kernel(logits_ref, val_ref, idx_ref, i_ref, max_ref, lidx_ref, *,
               stride, c0, block_c, nblocks):
    j = pl.program_id(0)
    nrows = logits_ref.shape[0]

    @pl.when(j == 0)
    def _init():
        row = lax.broadcasted_iota(jnp.uint32, (nrows, block_c), 0)
        colv = lax.broadcasted_iota(jnp.uint32, (nrows, block_c), 1)
        i_ref[...] = row * jnp.uint32(stride) + colv + jnp.uint32(c0)
        max_ref[...] = jnp.full_like(max_ref, -jnp.inf)
        lidx_ref[...] = jnp.zeros_like(lidx_ref)

    i = i_ref[...]
    u = _uniform_from_bits(_threefry_bits(i))
    phi = logits_ref[...] - jnp.log(-jnp.log(u))
    ilin = i.astype(jnp.int32)

    run_max = max_ref[...]
    run_idx = lidx_ref[...]
    for k in range(block_c // _LANES):
        sl = slice(k * _LANES, (k + 1) * _LANES)
        chunk = phi[:, sl]
        better = chunk > run_max
        run_max = jnp.where(better, chunk, run_max)
        run_idx = jnp.where(better, ilin[:, sl], run_idx)
    max_ref[...] = run_max
    lidx_ref[...] = run_idx
    i_ref[...] = i + jnp.uint32(block_c)

    @pl.when(j == nblocks - 1)
    def _finish():
        val_ref[...] = max_ref[...]
        idx_ref[...] = lidx_ref[...]


def _tc_partial(logits, c0, c1):
    nrows, ncols = logits.shape
    block_c = _BLOCK_C
    assert c0 % block_c == 0 and (c1 - c0) % block_c == 0
    j0 = c0 // block_c
    nblocks = (c1 - c0) // block_c
    return pl.pallas_call(
        functools.partial(_tc_kernel, stride=ncols, c0=c0,
                          block_c=block_c, nblocks=nblocks),
        grid=(nblocks,),
        in_specs=[pl.BlockSpec((nrows, block_c), lambda j: (0, j + j0))],
        out_specs=[pl.BlockSpec((nrows, _LANES), lambda j: (0, 0)),
                   pl.BlockSpec((nrows, _LANES), lambda j: (0, 0))],
        out_shape=[jax.ShapeDtypeStruct((nrows, _LANES), jnp.float32),
                   jax.ShapeDtypeStruct((nrows, _LANES), jnp.int32)],
        scratch_shapes=[
            pltpu.VMEM((nrows, block_c), jnp.uint32),
            pltpu.VMEM((nrows, _LANES), jnp.float32),
            pltpu.VMEM((nrows, _LANES), jnp.int32),
        ],
    )(logits)


def _sc_log2(x):
    b = lax.bitcast_convert_type(x, jnp.int32)
    e = (b >> jnp.int32(23)) - jnp.int32(127)
    m = lax.bitcast_convert_type(
        (b & jnp.int32(0x7FFFFF)) | jnp.int32(0x3F800000), jnp.float32)
    big = m >= jnp.float32(_SQRT2)
    m = jnp.where(big, m * jnp.float32(0.5), m)
    e = jnp.where(big, e + jnp.int32(1), e)
    z = (m - jnp.float32(1.0)) / (m + jnp.float32(1.0))
    z2 = z * z
    p = jnp.float32(_LOG2_C[5])
    for c in _LOG2_C[4::-1]:
        p = p * z2 + jnp.float32(c)
    return e.astype(jnp.float32) + z * p


def _sc_kernel(logits_hbm, val_hbm, idx_hbm, buf, vval, vidx, sem0, sem1,
               *, stride):
    c = lax.axis_index("c")
    s = lax.axis_index("s")
    wid = s * 2 + c
    band = wid % 8
    quarter = wid // 8
    qbase = quarter * _SC_QUARTER
    row0 = pl.multiple_of(band * 8, 8)
    lane = lax.iota(jnp.int32, 16).astype(jnp.uint32)

    def src(k):
        colstart = pl.multiple_of(qbase + k * _SC_CHUNK, _SC_CHUNK)
        return logits_hbm.at[pl.ds(row0, 8), pl.ds(colstart, _SC_CHUNK)]

    pltpu.async_copy(src(0), buf.at[0], sem0)

    def chunk_body(k, carry):
        p = k & 1
        colstart = pl.multiple_of(qbase + k * _SC_CHUNK, _SC_CHUNK)

        @pl.when((k + 1 < _SC_NCHUNKS) & (p == 0))
        def _start_odd():
            pltpu.async_copy(src(k + 1), buf.at[1], sem1)

        @pl.when((k + 1 < _SC_NCHUNKS) & (p == 1))
        def _start_even():
            pltpu.async_copy(src(k + 1), buf.at[0], sem0)

        @pl.when(p == 0)
        def _wait_even():
            pltpu.make_async_copy(src(k), buf.at[0], sem0).wait()

        @pl.when(p == 1)
        def _wait_odd():
            pltpu.make_async_copy(src(k), buf.at[1], sem1).wait()

        out = []
        for rr in range(8):
            run_max, run_idx = carry[2 * rr], carry[2 * rr + 1]
            ibase = ((row0 + rr) * stride + colstart).astype(jnp.uint32)

            def vec_body(v, c2, rr=rr, ibase=ibase):
                rm, ri = c2
                lvec = buf[p, rr, pl.ds(v * 16, 16)]
                i = ibase + (v * 16).astype(jnp.uint32) + lane
                u = _uniform_from_bits(_threefry_bits(i))
                inner = jnp.float32(-_LN2) * _sc_log2(u)
                phi = lvec - jnp.float32(_LN2) * _sc_log2(inner)
                better = phi > rm
                rm = jnp.where(better, phi, rm)
                ri = jnp.where(better, lax.bitcast_convert_type(i, jnp.int32), ri)
                return rm, ri

            run_max, run_idx = lax.fori_loop(
                0, _SC_CHUNK // 16, vec_body, (run_max, run_idx), unroll=4)
            out.extend([run_max, run_idx])
        return tuple(out)

    init = []
    for _ in range(8):
        init.extend([jnp.full((16,), -jnp.inf, jnp.float32),
                     jnp.zeros((16,), jnp.int32)])
    final = lax.fori_loop(0, _SC_NCHUNKS, chunk_body, tuple(init))
    for rr in range(8):
        vval[rr, :] = final[2 * rr]
        vidx[rr, :] = final[2 * rr + 1]
    pltpu.sync_copy(vval, val_hbm.at[wid])
    pltpu.sync_copy(vidx, idx_hbm.at[wid])


def _sc_partial(logits):
    nrows, ncols = logits.shape
    mesh = plsc.VectorSubcoreMesh(core_axis_name="c", subcore_axis_name="s",
                                  num_cores=2, num_subcores=16)
    return pl.kernel(
        functools.partial(_sc_kernel, stride=ncols),
        out_type=[jax.ShapeDtypeStruct((32, 8, 16), jnp.float32),
                  jax.ShapeDtypeStruct((32, 8, 16), jnp.int32)],
        mesh=mesh,
        scratch_types=[
            pltpu.VMEM((2, 8, _SC_CHUNK), jnp.float32),
            pltpu.VMEM((8, 16), jnp.float32),
            pltpu.VMEM((8, 16), jnp.int32),
            pltpu.SemaphoreType.DMA,
            pltpu.SemaphoreType.DMA,
        ],
    )(logits)


def _merge_kernel(tcv_ref, tci_ref, scv_ref, sci_ref, tail_ref, out_ref, *,
                  stride, tail0):
    nrows = tcv_ref.shape[0]
    ntail = tail_ref.shape[1]
    row = lax.broadcasted_iota(jnp.uint32, (nrows, ntail), 0)
    colv = lax.broadcasted_iota(jnp.uint32, (nrows, ntail), 1)
    ti = row * jnp.uint32(stride) + colv + jnp.uint32(tail0)
    u = _uniform_from_bits(_threefry_bits(ti))
    tphi = tail_ref[...] - jnp.log(-jnp.log(u))
    tlin = ti.astype(jnp.int32)
    tv = jnp.full((nrows, _LANES), -jnp.inf, jnp.float32)
    tix = jnp.zeros((nrows, _LANES), jnp.int32)
    for k in range(ntail // _LANES):
        sl = slice(k * _LANES, (k + 1) * _LANES)
        chunk = tphi[:, sl]
        better = chunk > tv
        tv = jnp.where(better, chunk, tv)
        tix = jnp.where(better, tlin[:, sl], tix)

    tcv = tcv_ref[...]
    tci = tci_ref[...]
    scv = scv_ref[...]
    sci = sci_ref[...]
    big = jnp.int32(0x7FFFFFFF)
    row_max = jnp.maximum(jnp.max(tcv, axis=1, keepdims=True),
                          jnp.max(scv, axis=1, keepdims=True))
    row_max = jnp.maximum(row_max, jnp.max(tv, axis=1, keepdims=True))
    c1 = jnp.min(jnp.where(tcv == row_max, tci, big), axis=1, keepdims=True)
    c2 = jnp.min(jnp.where(scv == row_max, sci, big), axis=1, keepdims=True)
    c3 = jnp.min(jnp.where(tv == row_max, tix, big), axis=1, keepdims=True)
    best_lin = jnp.minimum(jnp.minimum(c1, c2), c3)
    row1 = lax.broadcasted_iota(jnp.int32, (nrows, 1), 0)
    out_ref[...] = jnp.broadcast_to(best_lin - row1 * jnp.int32(stride),
                                    out_ref.shape)


def _merge(tcv, tci, scv, sci, tail, stride, tail0):
    nrows = tcv.shape[0]
    return pl.pallas_call(
        functools.partial(_merge_kernel, stride=stride, tail0=tail0),
        out_shape=jax.ShapeDtypeStruct((nrows, _LANES), jnp.int32),
    )(tcv, tci, scv, sci, tail)


def _sc_to_rows(x, fill):
    x = x.reshape(4, 8, 8, 16).transpose(1, 2, 0, 3).reshape(64, 64)
    pad = jnp.full((64, 64), fill, x.dtype)
    return jnp.concatenate([x, pad], axis=1)


@jax.jit
def kernel(logits):
    nrows, ncols = logits.shape
    c1 = (ncols // _BLOCK_C) * _BLOCK_C
    ntail_pad = ((ncols - c1 + _LANES - 1) // _LANES) * _LANES
    tail = jnp.pad(logits[:, c1:], ((0, 0), (0, ntail_pad - (ncols - c1))),
                   constant_values=-jnp.inf)
    tcv, tci = _tc_partial(logits, _SC_N, c1)
    scv, sci = _sc_partial(logits)
    scv_p = _sc_to_rows(scv, -jnp.inf)
    sci_p = _sc_to_rows(sci, 0x7FFFFFFF)
    out = _merge(tcv, tci, scv_p, sci_p, tail, ncols, c1)
    return out[:, :1]
